# baseline (speedup 1.0000x reference)
import functools

import jax
import jax.numpy as jnp
from jax import lax
from jax.experimental import pallas as pl
from jax.experimental.pallas import tpu as pltpu
from jax.experimental.pallas import tpu_sc as plsc

P = 16
V = 16
D = 16
N_CONT = 262144
N_CATEG = 262144
NDAYS = 3664

NC = 2
NS = 16
ROWS_T = N_CONT // NS
CHUNK = 2048
NCHUNK = ROWS_T // CHUNK
MB = CHUNK // 16

TWOPI = 6.283185307179586
INV2PI = 1.0 / TWOPI

SIN_C = (0.9999791148943297, -0.1666240153829831, 0.00830884993122673,
         -0.00019263169952744158, 2.147049615625063e-06)
COS_C = (0.9999992107412203, -0.4999942131500665, 0.04165977758594538,
         -0.0013858789204833017, 2.4202932054760706e-05,
         -2.1972921876445284e-07)

INV17 = tuple(10000.0 ** (-(2 * j) / 17.0) for j in range(9))
INV16 = tuple(10000.0 ** (-(2 * j) / 16.0) for j in range(8))


def _range_reduce(ang):
    q = (ang * INV2PI + 0.5).astype(jnp.int32).astype(jnp.float32)
    return ang - q * TWOPI


def _sin_poly(r, r2):
    s = jnp.float32(SIN_C[-1])
    for c in SIN_C[-2::-1]:
        s = s * r2 + c
    return s * r


def _cos_poly(r2):
    c = jnp.float32(COS_C[-1])
    for cc in COS_C[-2::-1]:
        c = c * r2 + cc
    return c


def _embed_body(pc_hbm, vals_hbm, dc_hbm, pg_hbm, vg_hbm, dg_hbm,
                pair_hbm, valtab_hbm, outc_hbm, outg_hbm,
                tab_v, pair_v, valtab_v, comb_v,
                b_p, b_d, b_vi, b_vf, out_v):
    cid = lax.axis_index("c")
    sid = lax.axis_index("s")
    base = sid * ROWS_T
    iota = lax.iota(jnp.int32, 16)

    pltpu.sync_copy(pair_hbm, pair_v)

    @pl.when(cid == 0)
    def _cont_side():
        def build(t, c0):
            d_f = (iota + t * 16).astype(jnp.float32)
            db = (iota + t * 16) * 17
            for j in range(8):
                r = _range_reduce(d_f * INV17[j])
                r2 = r * r
                plsc.store_scatter(tab_v, [db + (2 * j)], _sin_poly(r, r2))
                plsc.store_scatter(tab_v, [db + (2 * j + 1)], _cos_poly(r2))
            r = _range_reduce(d_f * INV17[8])
            plsc.store_scatter(tab_v, [db + 16], _sin_poly(r, r * r))
            return c0

        lax.fori_loop(0, NDAYS // 16, build, 0)

        def chunk_body(c, carry):
            off = base + c * CHUNK
            pltpu.sync_copy(pc_hbm.at[pl.ds(off, CHUNK)], b_p)
            pltpu.sync_copy(dc_hbm.at[pl.ds(off, CHUNK)], b_d)
            pltpu.sync_copy(vals_hbm.at[pl.ds(off, CHUNK)], b_vf)

            def mb_body(m, c2):
                sl = pl.ds(m * 16, 16)
                db = b_d[sl] * 17
                pb = b_p[sl] * D
                rowsb = iota * 17 + m * (16 * 17)
                for k in range(16):
                    pe = plsc.load_gather(tab_v, [db + k])
                    pr = plsc.load_gather(pair_v, [pb + k])
                    plsc.store_scatter(out_v, [rowsb + k], pe + pr)
                pe = plsc.load_gather(tab_v, [db + 16])
                plsc.store_scatter(out_v, [rowsb + 16], pe + b_vf[sl])
                return c2

            lax.fori_loop(0, MB, mb_body, 0)
            pltpu.sync_copy(out_v.at[pl.ds(0, CHUNK * 17)],
                            outc_hbm.at[pl.ds(off * 17, CHUNK * 17)])
            return carry

        lax.fori_loop(0, NCHUNK, chunk_body, 0)

    @pl.when(cid == 1)
    def _categ_side():
        pltpu.sync_copy(valtab_hbm, valtab_v)

        def build_comb(i, c0):
            sl = pl.ds(i * 16, 16)
            fl = iota + i * 16
            pk = ((fl >> 8) << 4) | (fl & 15)
            pr = plsc.load_gather(pair_v, [pk])
            comb_v[sl] = pr + valtab_v[sl]
            return c0

        lax.fori_loop(0, (P * V * D) // 16, build_comb, 0)

        def build(t, c0):
            d_f = (iota + t * 16).astype(jnp.float32)
            db = (iota + t * 16) * 16
            for j in range(8):
                r = _range_reduce(d_f * INV16[j])
                r2 = r * r
                plsc.store_scatter(tab_v, [db + (2 * j)], _sin_poly(r, r2))
                plsc.store_scatter(tab_v, [db + (2 * j + 1)], _cos_poly(r2))
            return c0

        lax.fori_loop(0, NDAYS // 16, build, 0)

        def chunk_body(c, carry):
            off = base + c * CHUNK
            pltpu.sync_copy(pg_hbm.at[pl.ds(off, CHUNK)], b_p)
            pltpu.sync_copy(dg_hbm.at[pl.ds(off, CHUNK)], b_d)
            pltpu.sync_copy(vg_hbm.at[pl.ds(off, CHUNK)], b_vi)

            def mb_body(m, c2):
                sl = pl.ds(m * 16, 16)
                db = b_d[sl] * 16
                cb = (b_p[sl] * 16 + b_vi[sl]) * 16
                rowsb = iota * 16 + m * (16 * 16)
                for k in range(16):
                    pe = plsc.load_gather(tab_v, [db + k])
                    cm = plsc.load_gather(comb_v, [cb + k])
                    plsc.store_scatter(out_v, [rowsb + k], pe + cm)
                return c2

            lax.fori_loop(0, MB, mb_body, 0)
            pltpu.sync_copy(out_v.at[pl.ds(0, CHUNK * 16)],
                            outg_hbm.at[pl.ds(off * 16, CHUNK * 16)])
            return carry

        lax.fori_loop(0, NCHUNK, chunk_body, 0)


@jax.jit
def kernel(ent_attr_ids_cont, vals_cont, days_cont,
           ent_attr_ids_categ, vocab_ids_categ, days_categ,
           pair_emb, categ_val_emb):
    mesh = plsc.VectorSubcoreMesh(core_axis_name="c", subcore_axis_name="s")
    f = pl.kernel(
        _embed_body,
        out_type=(jax.ShapeDtypeStruct((N_CONT * 17,), jnp.float32),
                  jax.ShapeDtypeStruct((N_CATEG * 16,), jnp.float32)),
        mesh=mesh,
        compiler_params=pltpu.CompilerParams(needs_layout_passes=False),
        scratch_types=[
            pltpu.VMEM((NDAYS * 17,), jnp.float32),
            pltpu.VMEM((P * D,), jnp.float32),
            pltpu.VMEM((P * V * D,), jnp.float32),
            pltpu.VMEM((P * V * D,), jnp.float32),
            pltpu.VMEM((CHUNK,), jnp.int32),
            pltpu.VMEM((CHUNK,), jnp.int32),
            pltpu.VMEM((CHUNK,), jnp.int32),
            pltpu.VMEM((CHUNK,), jnp.float32),
            pltpu.VMEM((CHUNK * 17,), jnp.float32),
        ],
    )
    outc, outg = f(ent_attr_ids_cont.astype(jnp.int32), vals_cont,
                   days_cont.astype(jnp.int32),
                   ent_attr_ids_categ.astype(jnp.int32),
                   vocab_ids_categ.astype(jnp.int32),
                   days_categ.astype(jnp.int32),
                   pair_emb.reshape(P * D), categ_val_emb.reshape(P * V * D))
    return outc.reshape(N_CONT, 17), outg.reshape(N_CATEG, 16)


# speedup vs baseline: 1.2806x; 1.2806x over previous
import functools

import jax
import jax.numpy as jnp
from jax import lax
from jax.experimental import pallas as pl
from jax.experimental.pallas import tpu as pltpu
from jax.experimental.pallas import tpu_sc as plsc

P = 16
V = 16
D = 16
N_CONT = 262144
N_CATEG = 262144
NDAYS = 3664

NC = 2
NS = 16
ROWS_T = N_CONT // NS
CHUNK = 2048
NCHUNK = ROWS_T // CHUNK
MB = CHUNK // 16

TWOPI = 6.283185307179586
INV2PI = 1.0 / TWOPI

SIN_C = (0.9999791148943297, -0.1666240153829831, 0.00830884993122673,
         -0.00019263169952744158, 2.147049615625063e-06)
COS_C = (0.9999992107412203, -0.4999942131500665, 0.04165977758594538,
         -0.0013858789204833017, 2.4202932054760706e-05,
         -2.1972921876445284e-07)

INV17 = tuple(10000.0 ** (-(2 * j) / 17.0) for j in range(9))
INV16 = tuple(10000.0 ** (-(2 * j) / 16.0) for j in range(8))


def _range_reduce(ang):
    q = (ang * INV2PI + 0.5).astype(jnp.int32).astype(jnp.float32)
    return ang - q * TWOPI


def _sin_poly(r, r2):
    s = jnp.float32(SIN_C[-1])
    for c in SIN_C[-2::-1]:
        s = s * r2 + c
    return s * r


def _cos_poly(r2):
    c = jnp.float32(COS_C[-1])
    for cc in COS_C[-2::-1]:
        c = c * r2 + cc
    return c


def _embed_body(pc_hbm, vals_hbm, dc_hbm, pg_hbm, vg_hbm, dg_hbm,
                pair_hbm, valtab_hbm, outc_hbm, outg_hbm,
                tab_v, pair_v, valtab_v, comb_v,
                b_p, b_d, b_vi, b_vf, out_v):
    cid = lax.axis_index("c")
    sid = lax.axis_index("s")
    base = sid * ROWS_T
    iota = lax.iota(jnp.int32, 16)

    pltpu.sync_copy(pair_hbm, pair_v)

    def _build_tab(inv, with_col16):
        def build(t, c0):
            d_f = (iota + t * 16).astype(jnp.float32)
            db = (iota + t * 16) * 17
            for j in range(8):
                r = _range_reduce(d_f * inv[j])
                r2 = r * r
                plsc.store_scatter(tab_v, [db + (2 * j)], _sin_poly(r, r2))
                plsc.store_scatter(tab_v, [db + (2 * j + 1)], _cos_poly(r2))
            if with_col16:
                r = _range_reduce(d_f * INV17[8])
                plsc.store_scatter(tab_v, [db + 16], _sin_poly(r, r * r))
            return c0

        lax.fori_loop(0, NDAYS // 16, build, 0)

    @pl.when(cid == 0)
    def _cont_side():
        _build_tab(INV17, True)

        def chunk_body(c, carry):
            off = base + c * CHUNK
            pltpu.sync_copy(pc_hbm.at[pl.ds(off, CHUNK)], b_p)
            pltpu.sync_copy(dc_hbm.at[pl.ds(off, CHUNK)], b_d)
            pltpu.sync_copy(vals_hbm.at[pl.ds(off, CHUNK)], b_vf)

            def mb_body(m, c2):
                sl = pl.ds(m * 16, 16)
                pv = b_p[sl]
                dv = b_d[sl]
                for r_i in range(16):
                    p = pv[r_i]
                    d = dv[r_i]
                    pr = pair_v[pl.ds(p * 16, 16)]
                    pe = tab_v[pl.ds(d * 17, 16)]
                    out_v[pl.ds((m * 16 + r_i) * 17, 16)] = pr + pe
                pe16 = plsc.load_gather(tab_v, [dv * 17 + 16])
                rows16 = iota * 17 + (m * (16 * 17) + 16)
                plsc.store_scatter(out_v, [rows16], pe16 + b_vf[sl])
                return c2

            lax.fori_loop(0, MB, mb_body, 0)
            pltpu.sync_copy(out_v.at[pl.ds(0, CHUNK * 17)],
                            outc_hbm.at[pl.ds(off * 17, CHUNK * 17)])
            return carry

        lax.fori_loop(0, NCHUNK, chunk_body, 0)

    @pl.when(cid == 1)
    def _categ_side():
        pltpu.sync_copy(valtab_hbm, valtab_v)

        def build_comb(i, c0):
            sl = pl.ds(i * 16, 16)
            fl = iota + i * 16
            pk = ((fl >> 8) << 4) | (fl & 15)
            pr = plsc.load_gather(pair_v, [pk])
            comb_v[sl] = pr + valtab_v[sl]
            return c0

        lax.fori_loop(0, (P * V * D) // 16, build_comb, 0)

        _build_tab(INV16, False)

        def chunk_body(c, carry):
            off = base + c * CHUNK
            pltpu.sync_copy(pg_hbm.at[pl.ds(off, CHUNK)], b_p)
            pltpu.sync_copy(dg_hbm.at[pl.ds(off, CHUNK)], b_d)
            pltpu.sync_copy(vg_hbm.at[pl.ds(off, CHUNK)], b_vi)

            def mb_body(m, c2):
                sl = pl.ds(m * 16, 16)
                pv = b_p[sl]
                vv = b_vi[sl]
                dv = b_d[sl]
                for r_i in range(16):
                    p = pv[r_i]
                    v = vv[r_i]
                    d = dv[r_i]
                    cm = comb_v[pl.ds((p * 16 + v) * 16, 16)]
                    pe = tab_v[pl.ds(d * 17, 16)]
                    out_v[pl.ds((m * 16 + r_i) * 16, 16)] = cm + pe
                return c2

            lax.fori_loop(0, MB, mb_body, 0)
            pltpu.sync_copy(out_v.at[pl.ds(0, CHUNK * 16)],
                            outg_hbm.at[pl.ds(off * 16, CHUNK * 16)])
            return carry

        lax.fori_loop(0, NCHUNK, chunk_body, 0)


@jax.jit
def kernel(ent_attr_ids_cont, vals_cont, days_cont,
           ent_attr_ids_categ, vocab_ids_categ, days_categ,
           pair_emb, categ_val_emb):
    mesh = plsc.VectorSubcoreMesh(core_axis_name="c", subcore_axis_name="s")
    f = pl.kernel(
        _embed_body,
        out_type=(jax.ShapeDtypeStruct((N_CONT * 17,), jnp.float32),
                  jax.ShapeDtypeStruct((N_CATEG * 16,), jnp.float32)),
        mesh=mesh,
        compiler_params=pltpu.CompilerParams(needs_layout_passes=False),
        scratch_types=[
            pltpu.VMEM((NDAYS * 17,), jnp.float32),
            pltpu.VMEM((P * D,), jnp.float32),
            pltpu.VMEM((P * V * D,), jnp.float32),
            pltpu.VMEM((P * V * D,), jnp.float32),
            pltpu.VMEM((CHUNK,), jnp.int32),
            pltpu.VMEM((CHUNK,), jnp.int32),
            pltpu.VMEM((CHUNK,), jnp.int32),
            pltpu.VMEM((CHUNK,), jnp.float32),
            pltpu.VMEM((CHUNK * 17,), jnp.float32),
        ],
    )
    outc, outg = f(ent_attr_ids_cont.astype(jnp.int32), vals_cont,
                   days_cont.astype(jnp.int32),
                   ent_attr_ids_categ.astype(jnp.int32),
                   vocab_ids_categ.astype(jnp.int32),
                   days_categ.astype(jnp.int32),
                   pair_emb.reshape(P * D), categ_val_emb.reshape(P * V * D))
    return outc.reshape(N_CONT, 17), outg.reshape(N_CATEG, 16)


# speedup vs baseline: 3.0570x; 2.3873x over previous
import functools

import jax
import jax.numpy as jnp
from jax import lax
from jax.experimental import pallas as pl
from jax.experimental.pallas import tpu as pltpu
from jax.experimental.pallas import tpu_sc as plsc

P = 16
V = 16
D = 16
N_CONT = 262144
N_CATEG = 262144
NDAYS = 3664

NC = 2
NS = 16
ROWS_T = N_CONT // NS
CHUNK = 2048
NCHUNK = ROWS_T // CHUNK
MB = CHUNK // 16
CSTRIDE = CHUNK + 8

TWOPI = 6.283185307179586
INV2PI = 1.0 / TWOPI

SIN_C = (0.9999791148943297, -0.1666240153829831, 0.00830884993122673,
         -0.00019263169952744158, 2.147049615625063e-06)
COS_C = (0.9999992107412203, -0.4999942131500665, 0.04165977758594538,
         -0.0013858789204833017, 2.4202932054760706e-05,
         -2.1972921876445284e-07)

INV17 = tuple(10000.0 ** (-(2 * j) / 17.0) for j in range(9))
INV16 = tuple(10000.0 ** (-(2 * j) / 16.0) for j in range(8))


def _range_reduce(ang):
    q = (ang * INV2PI + 0.5).astype(jnp.int32).astype(jnp.float32)
    return ang - q * TWOPI


def _sin_poly(r, r2):
    s = jnp.float32(SIN_C[-1])
    for c in SIN_C[-2::-1]:
        s = s * r2 + c
    return s * r


def _cos_poly(r2):
    c = jnp.float32(COS_C[-1])
    for cc in COS_C[-2::-1]:
        c = c * r2 + cc
    return c


def _embed_body(pc_hbm, vals_hbm, dc_hbm, pg_hbm, vg_hbm, dg_hbm,
                pair_hbm, valtab_hbm, outc_hbm, outg_hbm,
                tab_v, pair_v, valtab_v, comb_v,
                b_p, b_d, b_vi, b_vf, out_v, sem):
    cid = lax.axis_index("c")
    sid = lax.axis_index("s")
    base = sid * ROWS_T
    iota = lax.iota(jnp.int32, 16)
    kcol = iota * CSTRIDE

    pltpu.sync_copy(pair_hbm, pair_v)

    def _build_tab(inv, with_col16):
        def build(t, c0):
            d_f = (iota + t * 16).astype(jnp.float32)
            db = (iota + t * 16) * 17
            for j in range(8):
                r = _range_reduce(d_f * inv[j])
                r2 = r * r
                plsc.store_scatter(tab_v, [db + (2 * j)], _sin_poly(r, r2))
                plsc.store_scatter(tab_v, [db + (2 * j + 1)], _cos_poly(r2))
            if with_col16:
                r = _range_reduce(d_f * INV17[8])
                plsc.store_scatter(tab_v, [db + 16], _sin_poly(r, r * r))
            return c0

        lax.fori_loop(0, NDAYS // 16, build, 0)

    @pl.when(cid == 0)
    def _cont_side():
        _build_tab(INV17, True)

        def chunk_body(c, carry):
            off = base + c * CHUNK
            pltpu.sync_copy(pc_hbm.at[pl.ds(off, CHUNK)], b_p)
            pltpu.sync_copy(dc_hbm.at[pl.ds(off, CHUNK)], b_d)
            pltpu.sync_copy(vals_hbm.at[pl.ds(off, CHUNK)], b_vf)

            def mb_body(m, c2):
                sl = pl.ds(m * 16, 16)
                pv = b_p[sl]
                dv = b_d[sl]
                for r_i in range(16):
                    p = pv[r_i]
                    d = dv[r_i]
                    pr = pair_v[pl.ds(p * 16, 16)]
                    pe = tab_v[pl.ds(d * 17, 16)]
                    plsc.store_scatter(out_v, [kcol + (m * 16 + r_i)], pr + pe)
                pe16 = plsc.load_gather(tab_v, [dv * 17 + 16])
                out_v[pl.ds(16 * CSTRIDE + m * 16, 16)] = pe16 + b_vf[sl]
                return c2

            lax.fori_loop(0, MB, mb_body, 0)
            cps = [pltpu.async_copy(out_v.at[pl.ds(k * CSTRIDE, CHUNK)],
                                    outc_hbm.at[pl.ds(k * N_CONT + off, CHUNK)],
                                    sem)
                   for k in range(17)]
            for cp in cps:
                cp.wait()
            return carry

        lax.fori_loop(0, NCHUNK, chunk_body, 0)

    @pl.when(cid == 1)
    def _categ_side():
        pltpu.sync_copy(valtab_hbm, valtab_v)

        def build_comb(i, c0):
            sl = pl.ds(i * 16, 16)
            fl = iota + i * 16
            pk = ((fl >> 8) << 4) | (fl & 15)
            pr = plsc.load_gather(pair_v, [pk])
            comb_v[sl] = pr + valtab_v[sl]
            return c0

        lax.fori_loop(0, (P * V * D) // 16, build_comb, 0)

        _build_tab(INV16, False)

        def chunk_body(c, carry):
            off = base + c * CHUNK
            pltpu.sync_copy(pg_hbm.at[pl.ds(off, CHUNK)], b_p)
            pltpu.sync_copy(dg_hbm.at[pl.ds(off, CHUNK)], b_d)
            pltpu.sync_copy(vg_hbm.at[pl.ds(off, CHUNK)], b_vi)

            def mb_body(m, c2):
                sl = pl.ds(m * 16, 16)
                pv = b_p[sl]
                vv = b_vi[sl]
                dv = b_d[sl]
                for r_i in range(16):
                    p = pv[r_i]
                    v = vv[r_i]
                    d = dv[r_i]
                    cm = comb_v[pl.ds((p * 16 + v) * 16, 16)]
                    pe = tab_v[pl.ds(d * 17, 16)]
                    plsc.store_scatter(out_v, [kcol + (m * 16 + r_i)], cm + pe)
                return c2

            lax.fori_loop(0, MB, mb_body, 0)
            cps = [pltpu.async_copy(out_v.at[pl.ds(k * CSTRIDE, CHUNK)],
                                    outg_hbm.at[pl.ds(k * N_CATEG + off, CHUNK)],
                                    sem)
                   for k in range(16)]
            for cp in cps:
                cp.wait()
            return carry

        lax.fori_loop(0, NCHUNK, chunk_body, 0)


@jax.jit
def kernel(ent_attr_ids_cont, vals_cont, days_cont,
           ent_attr_ids_categ, vocab_ids_categ, days_categ,
           pair_emb, categ_val_emb):
    mesh = plsc.VectorSubcoreMesh(core_axis_name="c", subcore_axis_name="s")
    f = pl.kernel(
        _embed_body,
        out_type=(jax.ShapeDtypeStruct((17 * N_CONT,), jnp.float32),
                  jax.ShapeDtypeStruct((16 * N_CATEG,), jnp.float32)),
        mesh=mesh,
        compiler_params=pltpu.CompilerParams(needs_layout_passes=False),
        scratch_types=[
            pltpu.VMEM((NDAYS * 17,), jnp.float32),
            pltpu.VMEM((P * D,), jnp.float32),
            pltpu.VMEM((P * V * D,), jnp.float32),
            pltpu.VMEM((P * V * D,), jnp.float32),
            pltpu.VMEM((CHUNK,), jnp.int32),
            pltpu.VMEM((CHUNK,), jnp.int32),
            pltpu.VMEM((CHUNK,), jnp.int32),
            pltpu.VMEM((CHUNK,), jnp.float32),
            pltpu.VMEM((17 * CSTRIDE,), jnp.float32),
            pltpu.SemaphoreType.DMA,
        ],
    )
    outc, outg = f(ent_attr_ids_cont.astype(jnp.int32), vals_cont,
                   days_cont.astype(jnp.int32),
                   ent_attr_ids_categ.astype(jnp.int32),
                   vocab_ids_categ.astype(jnp.int32),
                   days_categ.astype(jnp.int32),
                   pair_emb.reshape(P * D), categ_val_emb.reshape(P * V * D))
    return outc.reshape(17, N_CONT).T, outg.reshape(16, N_CATEG).T
